# baseline (speedup 1.0000x reference)
import functools

import jax
import jax.numpy as jnp
from jax import lax
from jax.experimental import pallas as pl
from jax.experimental.pallas import tpu as pltpu
from jax.experimental.pallas import tpu_sc as plsc

H = 8
_NC, _NS = 2, 16
_NW = _NC * _NS


def _sc_mesh():
    return plsc.VectorSubcoreMesh(core_axis_name="c", subcore_axis_name="s",
                                  num_cores=_NC, num_subcores=_NS)


def _sc_gather2(tq, tkv, idx_q, idx_kv, chunk=80):
    n, wq = tq.shape
    wkv = tkv.shape[1]
    e = idx_q.shape[0]
    per_w = e // _NW
    nch = per_w // chunk

    @functools.partial(
        pl.kernel,
        out_type=[jax.ShapeDtypeStruct((e, wq), jnp.int32),
                  jax.ShapeDtypeStruct((e, wkv), jnp.int32)],
        mesh=_sc_mesh(),
        compiler_params=pltpu.CompilerParams(use_tc_tiling_on_sc=False),
        scratch_types=[
            pltpu.VMEM((per_w,), jnp.int32),
            pltpu.VMEM((per_w,), jnp.int32),
            pltpu.VMEM((2 * chunk, wq), jnp.int32),
            pltpu.VMEM((2 * chunk, wkv), jnp.int32),
            pltpu.SemaphoreType.DMA,
            pltpu.SemaphoreType.DMA,
        ],
    )
    def k(tq_hbm, tkv_hbm, iq_hbm, ikv_hbm, gq_hbm, gkv_hbm,
          idxq, idxkv, qrows, kvrows, gsem, wsem):
        wid = lax.axis_index("s") * _NC + lax.axis_index("c")
        base = wid * per_w
        pltpu.sync_copy(iq_hbm.at[pl.ds(base, per_w)], idxq)
        pltpu.sync_copy(ikv_hbm.at[pl.ds(base, per_w)], idxkv)

        def gathers(ci, buf):
            yield (tq_hbm.at[idxq.at[pl.ds(ci * chunk, chunk)]],
                   qrows.at[pl.ds(buf * chunk, chunk)])
            yield (tkv_hbm.at[idxkv.at[pl.ds(ci * chunk, chunk)]],
                   kvrows.at[pl.ds(buf * chunk, chunk)])

        def writes(ci, buf):
            yield (qrows.at[pl.ds(buf * chunk, chunk)],
                   gq_hbm.at[pl.ds(base + ci * chunk, chunk)])
            yield (kvrows.at[pl.ds(buf * chunk, chunk)],
                   gkv_hbm.at[pl.ds(base + ci * chunk, chunk)])

        def fire(ci, buf):
            for s, t in gathers(ci, buf):
                pltpu.async_copy(s, t, gsem)

        def drain_gather(ci, buf):
            for s, t in gathers(ci, buf):
                pltpu.make_async_copy(s, t, gsem).wait()

        def fire_write(ci, buf):
            for s, t in writes(ci, buf):
                pltpu.async_copy(s, t, wsem)

        def drain_write(ci, buf):
            for s, t in writes(ci, buf):
                pltpu.make_async_copy(s, t, wsem).wait()

        fire(0, 0)

        def body(ci, _):
            buf = lax.rem(ci, 2)
            nbuf = 1 - buf

            @pl.when(ci + 1 < nch)
            def _():
                @pl.when(ci >= 1)
                def _():
                    drain_write(ci - 1, nbuf)
                fire(ci + 1, nbuf)

            drain_gather(ci, buf)
            fire_write(ci, buf)
            return 0

        lax.fori_loop(0, nch, body, 0)
        if nch >= 2:
            drain_write(nch - 2, lax.rem(nch - 2, 2))
        drain_write(nch - 1, lax.rem(nch - 1, 2))

    return k(tq, tkv, idx_q, idx_kv)


def _sc_scatter_add(wex, dst3, zeros, n, chunk=80):
    e, w = wex.shape
    per_w = e // _NW
    nch = per_w // chunk
    nrow = (n // _NS) & ~7
    nlast = n - (_NS - 1) * nrow

    @functools.partial(
        pl.kernel,
        out_type=jax.ShapeDtypeStruct((_NC, n, w), jnp.float32),
        mesh=_sc_mesh(),
        compiler_params=pltpu.CompilerParams(use_tc_tiling_on_sc=False),
        scratch_types=[
            pltpu.VMEM((nch, chunk), jnp.int32),
            pltpu.VMEM((2 * chunk, w), jnp.float32),
            pltpu.VMEM_SHARED((n, w), jnp.float32),
            pltpu.SemaphoreType.DMA,
        ],
    )
    def k(wex_hbm, dst_hbm, zero_hbm, out_hbm, idxv, rows, table, lsem):
        cid = lax.axis_index("c")
        sid = lax.axis_index("s")
        wid = sid * _NC + cid
        base = wid * per_w
        @pl.when(sid < _NS - 1)
        def _():
            pltpu.sync_copy(zero_hbm.at[pl.ds(sid * nrow, nrow)],
                            table.at[pl.ds(sid * nrow, nrow)])

        @pl.when(sid == _NS - 1)
        def _():
            pltpu.sync_copy(zero_hbm.at[pl.ds((_NS - 1) * nrow, nlast)],
                            table.at[pl.ds((_NS - 1) * nrow, nlast)])

        pltpu.sync_copy(dst_hbm.at[wid], idxv)
        plsc.subcore_barrier()

        def fire(ci, buf):
            return pltpu.async_copy(
                wex_hbm.at[pl.ds(base + ci * chunk, chunk)],
                rows.at[pl.ds(buf * chunk, chunk)], lsem)

        def drain(ci, buf):
            pltpu.make_async_copy(
                wex_hbm.at[pl.ds(base + ci * chunk, chunk)],
                rows.at[pl.ds(buf * chunk, chunk)], lsem).wait()

        fire(0, 0)

        def body(ci, _):
            buf = lax.rem(ci, 2)

            @pl.when(ci + 1 < nch)
            def _():
                fire(ci + 1, 1 - buf)

            drain(ci, buf)
            pltpu.sync_copy(rows.at[pl.ds(buf * chunk, chunk)],
                            table.at[idxv.at[ci]], add=True)
            return 0

        lax.fori_loop(0, nch, body, 0)
        plsc.subcore_barrier()

        @pl.when(sid < _NS - 1)
        def _():
            pltpu.sync_copy(table.at[pl.ds(sid * nrow, nrow)],
                            out_hbm.at[cid, pl.ds(sid * nrow, nrow)])

        @pl.when(sid == _NS - 1)
        def _():
            pltpu.sync_copy(table.at[pl.ds((_NS - 1) * nrow, nlast)],
                            out_hbm.at[cid, pl.ds((_NS - 1) * nrow, nlast)])

    return k(wex, dst3, zeros)


def _ln(x, g, b, eps=1e-5):
    mu = x.mean(axis=-1, keepdims=True)
    var = ((x - mu) ** 2).mean(axis=-1, keepdims=True)
    return (x - mu) / jnp.sqrt(var + eps) * g + b


def _as_i32(a):
    m, w = a.shape
    return jax.lax.bitcast_convert_type(a.reshape(m, w // 2, 2), jnp.int32)


def _as_bf16(p):
    m, w2 = p.shape
    return jax.lax.bitcast_convert_type(p, jnp.bfloat16).reshape(m, 2 * w2)


def _node_pre_body(x_ref, wqT_ref, wkvT_ref, bq_ref, bkv_ref, g1_ref, b1_ref,
                   h_ref, qn_ref, knv_ref):
    x = x_ref[...]
    h = _ln(x, g1_ref[...], b1_ref[...])
    h_ref[...] = h
    qn = jnp.dot(h, wqT_ref[...], preferred_element_type=jnp.float32) + bq_ref[...]
    knv = jnp.dot(h, wkvT_ref[...], preferred_element_type=jnp.float32) + bkv_ref[...]
    qn_ref[...] = qn.astype(jnp.bfloat16)
    knv_ref[...] = knv.astype(jnp.bfloat16)


def _node_pre(x, wqT, wkvT, bq, bkv, g1, b1, bn):
    n, d = x.shape
    grid = (n // bn,)
    f32 = jnp.float32
    return pl.pallas_call(
        _node_pre_body,
        grid=grid,
        in_specs=[
            pl.BlockSpec((bn, d), lambda i: (i, 0)),
            pl.BlockSpec(wqT.shape, lambda i: (0, 0)),
            pl.BlockSpec(wkvT.shape, lambda i: (0, 0)),
            pl.BlockSpec((1, d), lambda i: (0, 0)),
            pl.BlockSpec((1, 2 * d), lambda i: (0, 0)),
            pl.BlockSpec((1, d), lambda i: (0, 0)),
            pl.BlockSpec((1, d), lambda i: (0, 0)),
        ],
        out_specs=[
            pl.BlockSpec((bn, d), lambda i: (i, 0)),
            pl.BlockSpec((bn, d), lambda i: (i, 0)),
            pl.BlockSpec((bn, 2 * d), lambda i: (i, 0)),
        ],
        out_shape=[
            jax.ShapeDtypeStruct((n, d), f32),
            jax.ShapeDtypeStruct((n, d), jnp.bfloat16),
            jax.ShapeDtypeStruct((n, 2 * d), jnp.bfloat16),
        ],
    )(x, wqT, wkvT, bq, bkv, g1, b1)


def _edge_body(gq_ref, gkv_ref, ea_ref, wkvT_ref, bkv_ref, red_ref, exp_ref,
               out_ref):
    d = gq_ref.shape[1]
    kv = jnp.dot(ea_ref[...], wkvT_ref[...], preferred_element_type=jnp.float32) + bkv_ref[...]
    ke = kv[:, :d]
    ve = kv[:, d:]
    gq = gq_ref[...].astype(jnp.float32)
    gkv = gkv_ref[...].astype(jnp.float32)
    prod = gq * (gkv[:, :d] + ke)
    alpha = jnp.dot(prod, red_ref[...], preferred_element_type=jnp.float32)
    ex = jnp.exp(alpha)
    exb = jnp.dot(ex, exp_ref[...], preferred_element_type=jnp.float32)
    w = (gkv[:, d:] + ve) * exb
    out_ref[...] = jnp.concatenate([w, ex], axis=1)


def _edge_stage(gq, gkv, ea, wkvT, bkv, red, expm, be):
    e, d = ea.shape
    grid = (e // be,)
    return pl.pallas_call(
        _edge_body,
        grid=grid,
        in_specs=[
            pl.BlockSpec((be, d), lambda i: (i, 0)),
            pl.BlockSpec((be, 2 * d), lambda i: (i, 0)),
            pl.BlockSpec((be, d), lambda i: (i, 0)),
            pl.BlockSpec(wkvT.shape, lambda i: (0, 0)),
            pl.BlockSpec((1, 2 * d), lambda i: (0, 0)),
            pl.BlockSpec(red.shape, lambda i: (0, 0)),
            pl.BlockSpec(expm.shape, lambda i: (0, 0)),
        ],
        out_specs=pl.BlockSpec((be, d + H), lambda i: (i, 0)),
        out_shape=jax.ShapeDtypeStruct((e, d + H), jnp.float32),
    )(gq, gkv, ea, wkvT, bkv, red, expm)


def _node_post_body(tbl_ref, x_ref, h_ref, expm_ref, wihT_ref, whhT_ref,
                    wselfT_ref, woutT_ref, b4_ref, g2_ref, b2_ref, wm1T_ref,
                    bm1_ref, wm2T_ref, bm2_ref, out_ref):
    d = x_ref.shape[1]
    t = jnp.sum(tbl_ref[...], axis=0)
    u = t[:, :d]
    den = t[:, d:]
    denb = jnp.dot(den, expm_ref[...], preferred_element_type=jnp.float32)
    agg = u / (denb + 1e-16)
    h = h_ref[...]
    b4 = b4_ref[...]
    gate = jax.nn.sigmoid(
        jnp.dot(agg, wihT_ref[...], preferred_element_type=jnp.float32)
        + jnp.dot(h, whhT_ref[...], preferred_element_type=jnp.float32)
        + b4[:, :d] + b4[:, d:2 * d])
    hs = jnp.dot(h, wselfT_ref[...], preferred_element_type=jnp.float32) + b4[:, 2 * d:3 * d]
    upd = agg + gate * (hs - agg)
    x1 = x_ref[...] + jnp.dot(upd, woutT_ref[...], preferred_element_type=jnp.float32) + b4[:, 3 * d:]
    h2 = _ln(x1, g2_ref[...], b2_ref[...])
    ff = jax.nn.relu(jnp.dot(h2, wm1T_ref[...], preferred_element_type=jnp.float32) + bm1_ref[...])
    ff = jnp.dot(ff, wm2T_ref[...], preferred_element_type=jnp.float32) + bm2_ref[...]
    out_ref[...] = x1 + ff


def _node_post(tbl, x, h, expm, wihT, whhT, wselfT, woutT, b4, g2, b2, wm1T,
               bm1, wm2T, bm2, bn):
    nsc, n, dh8 = tbl.shape
    d = x.shape[1]
    grid = (n // bn,)
    return pl.pallas_call(
        _node_post_body,
        grid=grid,
        in_specs=[
            pl.BlockSpec((nsc, bn, dh8), lambda i: (0, i, 0)),
            pl.BlockSpec((bn, d), lambda i: (i, 0)),
            pl.BlockSpec((bn, d), lambda i: (i, 0)),
            pl.BlockSpec(expm.shape, lambda i: (0, 0)),
            pl.BlockSpec(wihT.shape, lambda i: (0, 0)),
            pl.BlockSpec(whhT.shape, lambda i: (0, 0)),
            pl.BlockSpec(wselfT.shape, lambda i: (0, 0)),
            pl.BlockSpec(woutT.shape, lambda i: (0, 0)),
            pl.BlockSpec((1, 4 * d), lambda i: (0, 0)),
            pl.BlockSpec((1, d), lambda i: (0, 0)),
            pl.BlockSpec((1, d), lambda i: (0, 0)),
            pl.BlockSpec(wm1T.shape, lambda i: (0, 0)),
            pl.BlockSpec((1, 4 * d), lambda i: (0, 0)),
            pl.BlockSpec(wm2T.shape, lambda i: (0, 0)),
            pl.BlockSpec((1, d), lambda i: (0, 0)),
        ],
        out_specs=pl.BlockSpec((bn, d), lambda i: (i, 0)),
        out_shape=jax.ShapeDtypeStruct((n, d), jnp.float32),
    )(tbl, x, h, expm, wihT, whhT, wselfT, woutT, b4, g2, b2, wm1T, bm1, wm2T, bm2)


def kernel(x, edge_index, edge_attr, Wq, bq, Wkn, bkn, Wke, bke, Wvn, bvn,
           Wve, bve, Wself, bself, Wih, bih, Whh, bhh, Wout, bout, g1, b1,
           g2, b2, Wm1, bm1, Wm2, bm2):
    n, d = x.shape
    e = edge_index.shape[1]
    dh = d // H
    f32 = jnp.float32
    src = edge_index[0]
    dst = edge_index[1]

    row = lambda v: v.reshape(1, -1)
    eye = jnp.eye(H, dtype=f32)
    red = jnp.repeat(eye, dh, axis=0) * (1.0 / (dh ** 0.5))
    expm = jnp.repeat(eye, dh, axis=1)

    bn = 1000 if n % 1000 == 0 else n
    be = 2000 if e % 2000 == 0 else e

    h, qn, knv = _node_pre(
        x, Wq.T, jnp.concatenate([Wkn.T, Wvn.T], axis=1), row(bq),
        row(jnp.concatenate([bkn, bvn])), row(g1), row(b1), bn)

    gqp, gkvp = _sc_gather2(_as_i32(qn), _as_i32(knv), dst, src)
    gq = _as_bf16(gqp)
    gkv = _as_bf16(gkvp)

    wex = _edge_stage(gq, gkv, edge_attr, jnp.concatenate([Wke.T, Wve.T], axis=1),
                      row(jnp.concatenate([bke, bve])), red, expm, be)

    chunk = 80
    dst3 = dst.reshape(_NW, e // (_NW * chunk), chunk)
    zeros = jnp.zeros((n, d + H), f32)
    tbl = _sc_scatter_add(wex, dst3, zeros, n, chunk=chunk)

    out = _node_post(
        tbl, x, h, expm, Wih.T, Whh.T, Wself.T, Wout.T,
        row(jnp.concatenate([bih, bhh, bself, bout])), row(g2), row(b2),
        Wm1.T, row(bm1), Wm2.T, row(bm2), bn)
    return out


# speedup vs baseline: 2.8992x; 2.8992x over previous
import functools

import jax
import jax.numpy as jnp
from jax import lax
from jax.experimental import pallas as pl
from jax.experimental.pallas import tpu as pltpu
from jax.experimental.pallas import tpu_sc as plsc

H = 8
_NC, _NS = 2, 16
_NW = _NC * _NS


def _sc_mesh():
    return plsc.VectorSubcoreMesh(core_axis_name="c", subcore_axis_name="s",
                                  num_cores=_NC, num_subcores=_NS)


def _sc_gather2(tq, tkv, idx_q, idx_kv, chunk=80):
    n, wq = tq.shape
    wkv = tkv.shape[1]
    e = idx_q.shape[0]
    per_w = e // _NW
    nch = per_w // chunk

    @functools.partial(
        pl.kernel,
        out_type=[jax.ShapeDtypeStruct((e, wq), jnp.float32),
                  jax.ShapeDtypeStruct((e, wkv), jnp.int32)],
        mesh=_sc_mesh(),
        scratch_types=[
            pltpu.VMEM((per_w,), jnp.int32),
            pltpu.VMEM((per_w,), jnp.int32),
            pltpu.VMEM((2 * chunk, wq), jnp.float32),
            pltpu.VMEM((2 * chunk, wkv), jnp.int32),
            pltpu.SemaphoreType.DMA,
            pltpu.SemaphoreType.DMA,
        ],
    )
    def k(tq_hbm, tkv_hbm, iq_hbm, ikv_hbm, gq_hbm, gkv_hbm,
          idxq, idxkv, qrows, kvrows, gsem, wsem):
        wid = lax.axis_index("s") * _NC + lax.axis_index("c")
        base = wid * per_w
        pltpu.sync_copy(iq_hbm.at[pl.ds(base, per_w)], idxq)
        pltpu.sync_copy(ikv_hbm.at[pl.ds(base, per_w)], idxkv)

        def gathers(ci, buf):
            yield (tq_hbm.at[idxq.at[pl.ds(ci * chunk, chunk)]],
                   qrows.at[pl.ds(buf * chunk, chunk)])
            yield (tkv_hbm.at[idxkv.at[pl.ds(ci * chunk, chunk)]],
                   kvrows.at[pl.ds(buf * chunk, chunk)])

        def writes(ci, buf):
            yield (qrows.at[pl.ds(buf * chunk, chunk)],
                   gq_hbm.at[pl.ds(base + ci * chunk, chunk)])
            yield (kvrows.at[pl.ds(buf * chunk, chunk)],
                   gkv_hbm.at[pl.ds(base + ci * chunk, chunk)])

        def fire(ci, buf):
            for s, t in gathers(ci, buf):
                pltpu.async_copy(s, t, gsem)

        def drain_gather(ci, buf):
            for s, t in gathers(ci, buf):
                pltpu.make_async_copy(s, t, gsem).wait()

        def fire_write(ci, buf):
            for s, t in writes(ci, buf):
                pltpu.async_copy(s, t, wsem)

        def drain_write(ci, buf):
            for s, t in writes(ci, buf):
                pltpu.make_async_copy(s, t, wsem).wait()

        fire(0, 0)

        def body(ci, _):
            buf = lax.rem(ci, 2)
            nbuf = 1 - buf

            @pl.when(ci + 1 < nch)
            def _():
                @pl.when(ci >= 1)
                def _():
                    drain_write(ci - 1, nbuf)
                fire(ci + 1, nbuf)

            drain_gather(ci, buf)
            fire_write(ci, buf)
            return 0

        lax.fori_loop(0, nch, body, 0)
        if nch >= 2:
            drain_write(nch - 2, lax.rem(nch - 2, 2))
        drain_write(nch - 1, lax.rem(nch - 1, 2))

    return k(tq, tkv, idx_q, idx_kv)


def _sc_scatter_add(wex, dst3, zeros, n, chunk=80):
    e, w = wex.shape
    per_w = e // _NW
    nch = per_w // chunk
    nrow = (n // _NS) & ~7
    nlast = n - (_NS - 1) * nrow

    @functools.partial(
        pl.kernel,
        out_type=jax.ShapeDtypeStruct((_NC, n, w), jnp.float32),
        mesh=_sc_mesh(),
        compiler_params=pltpu.CompilerParams(use_tc_tiling_on_sc=False),
        scratch_types=[
            pltpu.VMEM((nch, chunk), jnp.int32),
            pltpu.VMEM((2 * chunk, w), jnp.float32),
            pltpu.VMEM_SHARED((n, w), jnp.float32),
            pltpu.SemaphoreType.DMA,
        ],
    )
    def k(wex_hbm, dst_hbm, zero_hbm, out_hbm, idxv, rows, table, lsem):
        cid = lax.axis_index("c")
        sid = lax.axis_index("s")
        wid = sid * _NC + cid
        base = wid * per_w
        @pl.when(sid < _NS - 1)
        def _():
            pltpu.sync_copy(zero_hbm.at[pl.ds(sid * nrow, nrow)],
                            table.at[pl.ds(sid * nrow, nrow)])

        @pl.when(sid == _NS - 1)
        def _():
            pltpu.sync_copy(zero_hbm.at[pl.ds((_NS - 1) * nrow, nlast)],
                            table.at[pl.ds((_NS - 1) * nrow, nlast)])

        pltpu.sync_copy(dst_hbm.at[wid], idxv)
        plsc.subcore_barrier()

        def fire(ci, buf):
            return pltpu.async_copy(
                wex_hbm.at[pl.ds(base + ci * chunk, chunk)],
                rows.at[pl.ds(buf * chunk, chunk)], lsem)

        def drain(ci, buf):
            pltpu.make_async_copy(
                wex_hbm.at[pl.ds(base + ci * chunk, chunk)],
                rows.at[pl.ds(buf * chunk, chunk)], lsem).wait()

        fire(0, 0)

        def body(ci, _):
            buf = lax.rem(ci, 2)

            @pl.when(ci + 1 < nch)
            def _():
                fire(ci + 1, 1 - buf)

            drain(ci, buf)
            pltpu.sync_copy(rows.at[pl.ds(buf * chunk, chunk)],
                            table.at[idxv.at[ci]], add=True)
            return 0

        lax.fori_loop(0, nch, body, 0)
        plsc.subcore_barrier()

        @pl.when(sid < _NS - 1)
        def _():
            pltpu.sync_copy(table.at[pl.ds(sid * nrow, nrow)],
                            out_hbm.at[cid, pl.ds(sid * nrow, nrow)])

        @pl.when(sid == _NS - 1)
        def _():
            pltpu.sync_copy(table.at[pl.ds((_NS - 1) * nrow, nlast)],
                            out_hbm.at[cid, pl.ds((_NS - 1) * nrow, nlast)])

    return k(wex, dst3, zeros)


def _ln(x, g, b, eps=1e-5):
    mu = x.mean(axis=-1, keepdims=True)
    var = ((x - mu) ** 2).mean(axis=-1, keepdims=True)
    return (x - mu) / jnp.sqrt(var + eps) * g + b


def _pack2(a, b):
    a16 = jax.lax.bitcast_convert_type(a.astype(jnp.bfloat16), jnp.uint16)
    b16 = jax.lax.bitcast_convert_type(b.astype(jnp.bfloat16), jnp.uint16)
    return (a16.astype(jnp.uint32)
            | (b16.astype(jnp.uint32) << 16)).astype(jnp.int32)


def _unpack2(w):
    wi = w.astype(jnp.uint32)
    a = jax.lax.bitcast_convert_type((wi << 16).astype(jnp.int32), jnp.float32)
    b = jax.lax.bitcast_convert_type(
        (wi & jnp.uint32(0xFFFF0000)).astype(jnp.int32), jnp.float32)
    return a, b


def _node_pre_body(x_ref, wqT_ref, wkvT_ref, bq_ref, bkv_ref, g1_ref, b1_ref,
                   h_ref, qn_ref, knv_ref):
    d = x_ref.shape[1]
    x = x_ref[...]
    h = _ln(x, g1_ref[...], b1_ref[...])
    h_ref[...] = h
    qn_ref[...] = jnp.dot(h, wqT_ref[...], preferred_element_type=jnp.float32) + bq_ref[...]
    knv = jnp.dot(h, wkvT_ref[...], preferred_element_type=jnp.float32) + bkv_ref[...]
    knv_ref[...] = _pack2(knv[:, :d], knv[:, d:])


def _node_pre(x, wqT, wkvT, bq, bkv, g1, b1, bn):
    n, d = x.shape
    grid = (n // bn,)
    f32 = jnp.float32
    return pl.pallas_call(
        _node_pre_body,
        grid=grid,
        in_specs=[
            pl.BlockSpec((bn, d), lambda i: (i, 0)),
            pl.BlockSpec(wqT.shape, lambda i: (0, 0)),
            pl.BlockSpec(wkvT.shape, lambda i: (0, 0)),
            pl.BlockSpec((1, d), lambda i: (0, 0)),
            pl.BlockSpec((1, 2 * d), lambda i: (0, 0)),
            pl.BlockSpec((1, d), lambda i: (0, 0)),
            pl.BlockSpec((1, d), lambda i: (0, 0)),
        ],
        out_specs=[
            pl.BlockSpec((bn, d), lambda i: (i, 0)),
            pl.BlockSpec((bn, d), lambda i: (i, 0)),
            pl.BlockSpec((bn, d), lambda i: (i, 0)),
        ],
        out_shape=[
            jax.ShapeDtypeStruct((n, d), f32),
            jax.ShapeDtypeStruct((n, d), f32),
            jax.ShapeDtypeStruct((n, d), jnp.int32),
        ],
    )(x, wqT, wkvT, bq, bkv, g1, b1)


def _edge_body(gq_ref, gkv_ref, ea_ref, wkvT_ref, bkv_ref, red_ref, exp_ref,
               out_ref):
    d = gq_ref.shape[1]
    kv = jnp.dot(ea_ref[...], wkvT_ref[...], preferred_element_type=jnp.float32) + bkv_ref[...]
    ke = kv[:, :d]
    ve = kv[:, d:]
    gkn, gvn = _unpack2(gkv_ref[...])
    prod = gq_ref[...] * (gkn + ke)
    alpha = jnp.dot(prod, red_ref[...], preferred_element_type=jnp.float32)
    ex = jnp.exp(alpha)
    exb = jnp.dot(ex, exp_ref[...], preferred_element_type=jnp.float32)
    w = (gvn + ve) * exb
    out_ref[...] = jnp.concatenate([w, ex], axis=1)


def _edge_stage(gq, gkv, ea, wkvT, bkv, red, expm, be):
    e, d = ea.shape
    grid = (e // be,)
    return pl.pallas_call(
        _edge_body,
        grid=grid,
        in_specs=[
            pl.BlockSpec((be, d), lambda i: (i, 0)),
            pl.BlockSpec((be, d), lambda i: (i, 0)),
            pl.BlockSpec((be, d), lambda i: (i, 0)),
            pl.BlockSpec(wkvT.shape, lambda i: (0, 0)),
            pl.BlockSpec((1, 2 * d), lambda i: (0, 0)),
            pl.BlockSpec(red.shape, lambda i: (0, 0)),
            pl.BlockSpec(expm.shape, lambda i: (0, 0)),
        ],
        out_specs=pl.BlockSpec((be, d + H), lambda i: (i, 0)),
        out_shape=jax.ShapeDtypeStruct((e, d + H), jnp.float32),
    )(gq, gkv, ea, wkvT, bkv, red, expm)


def _node_post_body(tbl_ref, x_ref, h_ref, expm_ref, wihT_ref, whhT_ref,
                    wselfT_ref, woutT_ref, b4_ref, g2_ref, b2_ref, wm1T_ref,
                    bm1_ref, wm2T_ref, bm2_ref, out_ref):
    d = x_ref.shape[1]
    t = jnp.sum(tbl_ref[...], axis=0)
    u = t[:, :d]
    den = t[:, d:]
    denb = jnp.dot(den, expm_ref[...], preferred_element_type=jnp.float32)
    agg = u / (denb + 1e-16)
    h = h_ref[...]
    b4 = b4_ref[...]
    gate = jax.nn.sigmoid(
        jnp.dot(agg, wihT_ref[...], preferred_element_type=jnp.float32)
        + jnp.dot(h, whhT_ref[...], preferred_element_type=jnp.float32)
        + b4[:, :d] + b4[:, d:2 * d])
    hs = jnp.dot(h, wselfT_ref[...], preferred_element_type=jnp.float32) + b4[:, 2 * d:3 * d]
    upd = agg + gate * (hs - agg)
    x1 = x_ref[...] + jnp.dot(upd, woutT_ref[...], preferred_element_type=jnp.float32) + b4[:, 3 * d:]
    h2 = _ln(x1, g2_ref[...], b2_ref[...])
    ff = jax.nn.relu(jnp.dot(h2, wm1T_ref[...], preferred_element_type=jnp.float32) + bm1_ref[...])
    ff = jnp.dot(ff, wm2T_ref[...], preferred_element_type=jnp.float32) + bm2_ref[...]
    out_ref[...] = x1 + ff


def _node_post(tbl, x, h, expm, wihT, whhT, wselfT, woutT, b4, g2, b2, wm1T,
               bm1, wm2T, bm2, bn):
    nsc, n, dh8 = tbl.shape
    d = x.shape[1]
    grid = (n // bn,)
    return pl.pallas_call(
        _node_post_body,
        grid=grid,
        in_specs=[
            pl.BlockSpec((nsc, bn, dh8), lambda i: (0, i, 0)),
            pl.BlockSpec((bn, d), lambda i: (i, 0)),
            pl.BlockSpec((bn, d), lambda i: (i, 0)),
            pl.BlockSpec(expm.shape, lambda i: (0, 0)),
            pl.BlockSpec(wihT.shape, lambda i: (0, 0)),
            pl.BlockSpec(whhT.shape, lambda i: (0, 0)),
            pl.BlockSpec(wselfT.shape, lambda i: (0, 0)),
            pl.BlockSpec(woutT.shape, lambda i: (0, 0)),
            pl.BlockSpec((1, 4 * d), lambda i: (0, 0)),
            pl.BlockSpec((1, d), lambda i: (0, 0)),
            pl.BlockSpec((1, d), lambda i: (0, 0)),
            pl.BlockSpec(wm1T.shape, lambda i: (0, 0)),
            pl.BlockSpec((1, 4 * d), lambda i: (0, 0)),
            pl.BlockSpec(wm2T.shape, lambda i: (0, 0)),
            pl.BlockSpec((1, d), lambda i: (0, 0)),
        ],
        out_specs=pl.BlockSpec((bn, d), lambda i: (i, 0)),
        out_shape=jax.ShapeDtypeStruct((n, d), jnp.float32),
    )(tbl, x, h, expm, wihT, whhT, wselfT, woutT, b4, g2, b2, wm1T, bm1, wm2T, bm2)


def kernel(x, edge_index, edge_attr, Wq, bq, Wkn, bkn, Wke, bke, Wvn, bvn,
           Wve, bve, Wself, bself, Wih, bih, Whh, bhh, Wout, bout, g1, b1,
           g2, b2, Wm1, bm1, Wm2, bm2):
    n, d = x.shape
    e = edge_index.shape[1]
    dh = d // H
    f32 = jnp.float32
    src = edge_index[0]
    dst = edge_index[1]

    row = lambda v: v.reshape(1, -1)
    eye = jnp.eye(H, dtype=f32)
    red = jnp.repeat(eye, dh, axis=0) * (1.0 / (dh ** 0.5))
    expm = jnp.repeat(eye, dh, axis=1)

    bn = 1000 if n % 1000 == 0 else n
    be = 2000 if e % 2000 == 0 else e

    h, qn, knv = _node_pre(
        x, Wq.T, jnp.concatenate([Wkn.T, Wvn.T], axis=1), row(bq),
        row(jnp.concatenate([bkn, bvn])), row(g1), row(b1), bn)

    gq, gkv = _sc_gather2(qn, knv, dst, src)

    wex = _edge_stage(gq, gkv, edge_attr, jnp.concatenate([Wke.T, Wve.T], axis=1),
                      row(jnp.concatenate([bke, bve])), red, expm, be)

    chunk = 80
    dst3 = dst.reshape(_NW, e // (_NW * chunk), chunk)
    zeros = jnp.zeros((n, d + H), f32)
    tbl = _sc_scatter_add(wex, dst3, zeros, n, chunk=chunk)

    out = _node_post(
        tbl, x, h, expm, Wih.T, Whh.T, Wself.T, Wout.T,
        row(jnp.concatenate([bih, bhh, bself, bout])), row(g2), row(b2),
        Wm1.T, row(bm1), Wm2.T, row(bm2), bn)
    return out


# speedup vs baseline: 3.0738x; 1.0602x over previous
import functools

import jax
import jax.numpy as jnp
from jax import lax
from jax.experimental import pallas as pl
from jax.experimental.pallas import tpu as pltpu
from jax.experimental.pallas import tpu_sc as plsc

H = 8
_NC, _NS = 2, 16
_NW = _NC * _NS


def _sc_mesh():
    return plsc.VectorSubcoreMesh(core_axis_name="c", subcore_axis_name="s",
                                  num_cores=_NC, num_subcores=_NS)


def _sc_gather2(tq, tkv, idx_q, idx_kv, chunk=80):
    n, wq = tq.shape
    wkv = tkv.shape[1]
    e = idx_q.shape[0]
    per_w = e // _NW
    nch = per_w // chunk

    @functools.partial(
        pl.kernel,
        out_type=[jax.ShapeDtypeStruct((e, wq), jnp.float32),
                  jax.ShapeDtypeStruct((e, wkv), jnp.int32)],
        mesh=_sc_mesh(),
        scratch_types=[
            pltpu.VMEM((per_w,), jnp.int32),
            pltpu.VMEM((per_w,), jnp.int32),
            pltpu.VMEM((4 * chunk, wq), jnp.float32),
            pltpu.VMEM((4 * chunk, wkv), jnp.int32),
            pltpu.SemaphoreType.DMA,
            pltpu.SemaphoreType.DMA,
        ],
    )
    def k(tq_hbm, tkv_hbm, iq_hbm, ikv_hbm, gq_hbm, gkv_hbm,
          idxq, idxkv, qrows, kvrows, gsem, wsem):
        wid = lax.axis_index("s") * _NC + lax.axis_index("c")
        base = wid * per_w
        pltpu.sync_copy(iq_hbm.at[pl.ds(base, per_w)], idxq)
        pltpu.sync_copy(ikv_hbm.at[pl.ds(base, per_w)], idxkv)

        def gathers(ci, buf):
            yield (tq_hbm.at[idxq.at[pl.ds(ci * chunk, chunk)]],
                   qrows.at[pl.ds(buf * chunk, chunk)])
            yield (tkv_hbm.at[idxkv.at[pl.ds(ci * chunk, chunk)]],
                   kvrows.at[pl.ds(buf * chunk, chunk)])

        def writes(ci, buf):
            yield (qrows.at[pl.ds(buf * chunk, chunk)],
                   gq_hbm.at[pl.ds(base + ci * chunk, chunk)])
            yield (kvrows.at[pl.ds(buf * chunk, chunk)],
                   gkv_hbm.at[pl.ds(base + ci * chunk, chunk)])

        def fire(ci, buf):
            for s, t in gathers(ci, buf):
                pltpu.async_copy(s, t, gsem)

        def drain_gather(ci, buf):
            for s, t in gathers(ci, buf):
                pltpu.make_async_copy(s, t, gsem).wait()

        def fire_write(ci, buf):
            for s, t in writes(ci, buf):
                pltpu.async_copy(s, t, wsem)

        def drain_write(ci, buf):
            for s, t in writes(ci, buf):
                pltpu.make_async_copy(s, t, wsem).wait()

        for j in range(min(3, nch)):
            fire(j, j)

        def body(ci, _):
            buf = lax.rem(ci, 4)
            pre = ci + 3
            pbuf = lax.rem(pre, 4)

            @pl.when(pre < nch)
            def _():
                @pl.when(ci >= 1)
                def _():
                    drain_write(ci - 1, lax.rem(ci - 1, 4))
                fire(pre, pbuf)

            drain_gather(ci, buf)
            fire_write(ci, buf)
            return 0

        lax.fori_loop(0, nch, body, 0)
        for j in range(max(0, nch - 4), nch):
            drain_write(j, j % 4)

    return k(tq, tkv, idx_q, idx_kv)


def _sc_scatter_add(wex, dst3, zeros, n, chunk=80):
    e, w = wex.shape
    per_w = e // _NW
    nch = per_w // chunk
    nrow = (n // _NS) & ~7
    nlast = n - (_NS - 1) * nrow

    @functools.partial(
        pl.kernel,
        out_type=jax.ShapeDtypeStruct((_NC, n, w), jnp.float32),
        mesh=_sc_mesh(),
        compiler_params=pltpu.CompilerParams(use_tc_tiling_on_sc=False),
        scratch_types=[
            pltpu.VMEM((nch, chunk), jnp.int32),
            pltpu.VMEM((3 * chunk, w), jnp.float32),
            pltpu.VMEM_SHARED((n, w), jnp.float32),
            pltpu.SemaphoreType.DMA,
            pltpu.SemaphoreType.DMA,
        ],
    )
    def k(wex_hbm, dst_hbm, zero_hbm, out_hbm, idxv, rows, table, lsem, ssem):
        cid = lax.axis_index("c")
        sid = lax.axis_index("s")
        wid = sid * _NC + cid
        base = wid * per_w
        @pl.when(sid < _NS - 1)
        def _():
            pltpu.sync_copy(zero_hbm.at[pl.ds(sid * nrow, nrow)],
                            table.at[pl.ds(sid * nrow, nrow)])

        @pl.when(sid == _NS - 1)
        def _():
            pltpu.sync_copy(zero_hbm.at[pl.ds((_NS - 1) * nrow, nlast)],
                            table.at[pl.ds((_NS - 1) * nrow, nlast)])

        pltpu.sync_copy(dst_hbm.at[wid], idxv)
        plsc.subcore_barrier()

        def fire(ci, buf):
            return pltpu.async_copy(
                wex_hbm.at[pl.ds(base + ci * chunk, chunk)],
                rows.at[pl.ds(buf * chunk, chunk)], lsem)

        def drain(ci, buf):
            pltpu.make_async_copy(
                wex_hbm.at[pl.ds(base + ci * chunk, chunk)],
                rows.at[pl.ds(buf * chunk, chunk)], lsem).wait()

        def fire_scat(ci, buf):
            return pltpu.async_copy(rows.at[pl.ds(buf * chunk, chunk)],
                                    table.at[idxv.at[ci]], ssem, add=True)

        def drain_scat(ci, buf):
            pltpu.make_async_copy(rows.at[pl.ds(buf * chunk, chunk)],
                                  table.at[idxv.at[ci]], ssem).wait()

        for j in range(min(2, nch)):
            fire(j, j)

        def body(ci, _):
            buf = lax.rem(ci, 3)
            pre = ci + 2

            @pl.when(pre < nch)
            def _():
                @pl.when(ci >= 1)
                def _():
                    drain_scat(ci - 1, lax.rem(ci - 1, 3))
                fire(pre, lax.rem(pre, 3))

            drain(ci, buf)
            fire_scat(ci, buf)
            return 0

        lax.fori_loop(0, nch, body, 0)
        for j in range(max(0, nch - 3), nch):
            drain_scat(j, j % 3)
        plsc.subcore_barrier()

        @pl.when(sid < _NS - 1)
        def _():
            pltpu.sync_copy(table.at[pl.ds(sid * nrow, nrow)],
                            out_hbm.at[cid, pl.ds(sid * nrow, nrow)])

        @pl.when(sid == _NS - 1)
        def _():
            pltpu.sync_copy(table.at[pl.ds((_NS - 1) * nrow, nlast)],
                            out_hbm.at[cid, pl.ds((_NS - 1) * nrow, nlast)])

    return k(wex, dst3, zeros)


def _ln(x, g, b, eps=1e-5):
    mu = x.mean(axis=-1, keepdims=True)
    var = ((x - mu) ** 2).mean(axis=-1, keepdims=True)
    return (x - mu) / jnp.sqrt(var + eps) * g + b


def _pack2(a, b):
    a16 = jax.lax.bitcast_convert_type(a.astype(jnp.bfloat16), jnp.uint16)
    b16 = jax.lax.bitcast_convert_type(b.astype(jnp.bfloat16), jnp.uint16)
    return (a16.astype(jnp.uint32)
            | (b16.astype(jnp.uint32) << 16)).astype(jnp.int32)


def _unpack2(w):
    wi = w.astype(jnp.uint32)
    a = jax.lax.bitcast_convert_type((wi << 16).astype(jnp.int32), jnp.float32)
    b = jax.lax.bitcast_convert_type(
        (wi & jnp.uint32(0xFFFF0000)).astype(jnp.int32), jnp.float32)
    return a, b


def _node_pre_body(x_ref, wqT_ref, wkvT_ref, bq_ref, bkv_ref, g1_ref, b1_ref,
                   h_ref, qn_ref, knv_ref):
    d = x_ref.shape[1]
    x = x_ref[...]
    h = _ln(x, g1_ref[...], b1_ref[...])
    h_ref[...] = h
    qn_ref[...] = jnp.dot(h, wqT_ref[...], preferred_element_type=jnp.float32) + bq_ref[...]
    knv = jnp.dot(h, wkvT_ref[...], preferred_element_type=jnp.float32) + bkv_ref[...]
    knv_ref[...] = _pack2(knv[:, :d], knv[:, d:])


def _node_pre(x, wqT, wkvT, bq, bkv, g1, b1, bn):
    n, d = x.shape
    grid = (n // bn,)
    f32 = jnp.float32
    return pl.pallas_call(
        _node_pre_body,
        grid=grid,
        in_specs=[
            pl.BlockSpec((bn, d), lambda i: (i, 0)),
            pl.BlockSpec(wqT.shape, lambda i: (0, 0)),
            pl.BlockSpec(wkvT.shape, lambda i: (0, 0)),
            pl.BlockSpec((1, d), lambda i: (0, 0)),
            pl.BlockSpec((1, 2 * d), lambda i: (0, 0)),
            pl.BlockSpec((1, d), lambda i: (0, 0)),
            pl.BlockSpec((1, d), lambda i: (0, 0)),
        ],
        out_specs=[
            pl.BlockSpec((bn, d), lambda i: (i, 0)),
            pl.BlockSpec((bn, d), lambda i: (i, 0)),
            pl.BlockSpec((bn, d), lambda i: (i, 0)),
        ],
        out_shape=[
            jax.ShapeDtypeStruct((n, d), f32),
            jax.ShapeDtypeStruct((n, d), f32),
            jax.ShapeDtypeStruct((n, d), jnp.int32),
        ],
    )(x, wqT, wkvT, bq, bkv, g1, b1)


def _edge_body(gq_ref, gkv_ref, ea_ref, wkvT_ref, bkv_ref, red_ref, exp_ref,
               out_ref):
    d = gq_ref.shape[1]
    kv = jnp.dot(ea_ref[...], wkvT_ref[...], preferred_element_type=jnp.float32) + bkv_ref[...]
    ke = kv[:, :d]
    ve = kv[:, d:]
    gkn, gvn = _unpack2(gkv_ref[...])
    prod = gq_ref[...] * (gkn + ke)
    alpha = jnp.dot(prod, red_ref[...], preferred_element_type=jnp.float32)
    ex = jnp.exp(alpha)
    exb = jnp.dot(ex, exp_ref[...], preferred_element_type=jnp.float32)
    w = (gvn + ve) * exb
    out_ref[...] = jnp.concatenate([w, ex], axis=1)


def _edge_stage(gq, gkv, ea, wkvT, bkv, red, expm, be):
    e, d = ea.shape
    grid = (e // be,)
    return pl.pallas_call(
        _edge_body,
        grid=grid,
        in_specs=[
            pl.BlockSpec((be, d), lambda i: (i, 0)),
            pl.BlockSpec((be, d), lambda i: (i, 0)),
            pl.BlockSpec((be, d), lambda i: (i, 0)),
            pl.BlockSpec(wkvT.shape, lambda i: (0, 0)),
            pl.BlockSpec((1, 2 * d), lambda i: (0, 0)),
            pl.BlockSpec(red.shape, lambda i: (0, 0)),
            pl.BlockSpec(expm.shape, lambda i: (0, 0)),
        ],
        out_specs=pl.BlockSpec((be, d + H), lambda i: (i, 0)),
        out_shape=jax.ShapeDtypeStruct((e, d + H), jnp.float32),
    )(gq, gkv, ea, wkvT, bkv, red, expm)


def _node_post_body(tbl_ref, x_ref, h_ref, expm_ref, wihT_ref, whhT_ref,
                    wselfT_ref, woutT_ref, b4_ref, g2_ref, b2_ref, wm1T_ref,
                    bm1_ref, wm2T_ref, bm2_ref, out_ref):
    d = x_ref.shape[1]
    t = jnp.sum(tbl_ref[...], axis=0)
    u = t[:, :d]
    den = t[:, d:]
    denb = jnp.dot(den, expm_ref[...], preferred_element_type=jnp.float32)
    agg = u / (denb + 1e-16)
    h = h_ref[...]
    b4 = b4_ref[...]
    gate = jax.nn.sigmoid(
        jnp.dot(agg, wihT_ref[...], preferred_element_type=jnp.float32)
        + jnp.dot(h, whhT_ref[...], preferred_element_type=jnp.float32)
        + b4[:, :d] + b4[:, d:2 * d])
    hs = jnp.dot(h, wselfT_ref[...], preferred_element_type=jnp.float32) + b4[:, 2 * d:3 * d]
    upd = agg + gate * (hs - agg)
    x1 = x_ref[...] + jnp.dot(upd, woutT_ref[...], preferred_element_type=jnp.float32) + b4[:, 3 * d:]
    h2 = _ln(x1, g2_ref[...], b2_ref[...])
    ff = jax.nn.relu(jnp.dot(h2, wm1T_ref[...], preferred_element_type=jnp.float32) + bm1_ref[...])
    ff = jnp.dot(ff, wm2T_ref[...], preferred_element_type=jnp.float32) + bm2_ref[...]
    out_ref[...] = x1 + ff


def _node_post(tbl, x, h, expm, wihT, whhT, wselfT, woutT, b4, g2, b2, wm1T,
               bm1, wm2T, bm2, bn):
    nsc, n, dh8 = tbl.shape
    d = x.shape[1]
    grid = (n // bn,)
    return pl.pallas_call(
        _node_post_body,
        grid=grid,
        in_specs=[
            pl.BlockSpec((nsc, bn, dh8), lambda i: (0, i, 0)),
            pl.BlockSpec((bn, d), lambda i: (i, 0)),
            pl.BlockSpec((bn, d), lambda i: (i, 0)),
            pl.BlockSpec(expm.shape, lambda i: (0, 0)),
            pl.BlockSpec(wihT.shape, lambda i: (0, 0)),
            pl.BlockSpec(whhT.shape, lambda i: (0, 0)),
            pl.BlockSpec(wselfT.shape, lambda i: (0, 0)),
            pl.BlockSpec(woutT.shape, lambda i: (0, 0)),
            pl.BlockSpec((1, 4 * d), lambda i: (0, 0)),
            pl.BlockSpec((1, d), lambda i: (0, 0)),
            pl.BlockSpec((1, d), lambda i: (0, 0)),
            pl.BlockSpec(wm1T.shape, lambda i: (0, 0)),
            pl.BlockSpec((1, 4 * d), lambda i: (0, 0)),
            pl.BlockSpec(wm2T.shape, lambda i: (0, 0)),
            pl.BlockSpec((1, d), lambda i: (0, 0)),
        ],
        out_specs=pl.BlockSpec((bn, d), lambda i: (i, 0)),
        out_shape=jax.ShapeDtypeStruct((n, d), jnp.float32),
    )(tbl, x, h, expm, wihT, whhT, wselfT, woutT, b4, g2, b2, wm1T, bm1, wm2T, bm2)


def kernel(x, edge_index, edge_attr, Wq, bq, Wkn, bkn, Wke, bke, Wvn, bvn,
           Wve, bve, Wself, bself, Wih, bih, Whh, bhh, Wout, bout, g1, b1,
           g2, b2, Wm1, bm1, Wm2, bm2):
    n, d = x.shape
    e = edge_index.shape[1]
    dh = d // H
    f32 = jnp.float32
    src = edge_index[0]
    dst = edge_index[1]

    row = lambda v: v.reshape(1, -1)
    eye = jnp.eye(H, dtype=f32)
    red = jnp.repeat(eye, dh, axis=0) * (1.0 / (dh ** 0.5))
    expm = jnp.repeat(eye, dh, axis=1)

    bn = 1000 if n % 1000 == 0 else n
    be = 2000 if e % 2000 == 0 else e

    h, qn, knv = _node_pre(
        x, Wq.T, jnp.concatenate([Wkn.T, Wvn.T], axis=1), row(bq),
        row(jnp.concatenate([bkn, bvn])), row(g1), row(b1), bn)

    gq, gkv = _sc_gather2(qn, knv, dst, src)

    wex = _edge_stage(gq, gkv, edge_attr, jnp.concatenate([Wke.T, Wve.T], axis=1),
                      row(jnp.concatenate([bke, bve])), red, expm, be)

    chunk = 80
    dst3 = dst.reshape(_NW, e // (_NW * chunk), chunk)
    zeros = jnp.zeros((n, d + H), f32)
    tbl = _sc_scatter_add(wex, dst3, zeros, n, chunk=chunk)

    out = _node_post(
        tbl, x, h, expm, Wih.T, Whh.T, Wself.T, Wout.T,
        row(jnp.concatenate([bih, bhh, bself, bout])), row(g2), row(b2),
        Wm1.T, row(bm1), Wm2.T, row(bm2), bn)
    return out


# speedup vs baseline: 3.1158x; 1.0137x over previous
import functools

import jax
import jax.numpy as jnp
from jax import lax
from jax.experimental import pallas as pl
from jax.experimental.pallas import tpu as pltpu
from jax.experimental.pallas import tpu_sc as plsc

H = 8
_NC, _NS = 2, 16
_NW = _NC * _NS


def _sc_mesh():
    return plsc.VectorSubcoreMesh(core_axis_name="c", subcore_axis_name="s",
                                  num_cores=_NC, num_subcores=_NS)


def _split_edges(ei, bs=None):
    e = ei.shape[1]
    bs = e

    def body(ei_ref, s_ref, d_ref):
        s_ref[...] = ei_ref[0, :]
        d_ref[...] = ei_ref[1, :]

    return pl.pallas_call(
        body,
        grid=(e // bs,),
        in_specs=[pl.BlockSpec((2, bs), lambda i: (0, i))],
        out_specs=[pl.BlockSpec((bs,), lambda i: (i,)),
                   pl.BlockSpec((bs,), lambda i: (i,))],
        out_shape=[jax.ShapeDtypeStruct((e,), jnp.int32),
                   jax.ShapeDtypeStruct((e,), jnp.int32)],
    )(ei)


def _sc_gather2(tq, tkv, src, dst, chunk=80):
    n, wq = tq.shape
    wkv = tkv.shape[1]
    e = src.shape[0]
    per_w = e // _NW
    nch = per_w // chunk

    @functools.partial(
        pl.kernel,
        out_type=[jax.ShapeDtypeStruct((e, wq), jnp.float32),
                  jax.ShapeDtypeStruct((e, wkv), jnp.int32)],
        mesh=_sc_mesh(),
        scratch_types=[
            pltpu.VMEM((per_w,), jnp.int32),
            pltpu.VMEM((per_w,), jnp.int32),
            pltpu.VMEM((4 * chunk, wq), jnp.float32),
            pltpu.VMEM((4 * chunk, wkv), jnp.int32),
            pltpu.SemaphoreType.DMA,
            pltpu.SemaphoreType.DMA,
        ],
    )
    def k(tq_hbm, tkv_hbm, iq_hbm, ikv_hbm, gq_hbm, gkv_hbm,
          idxq, idxkv, qrows, kvrows, gsem, wsem):
        wid = lax.axis_index("s") * _NC + lax.axis_index("c")
        base = wid * per_w
        pltpu.sync_copy(iq_hbm.at[pl.ds(base, per_w)], idxq)
        pltpu.sync_copy(ikv_hbm.at[pl.ds(base, per_w)], idxkv)

        def gathers(ci, buf):
            yield (tq_hbm.at[idxq.at[pl.ds(ci * chunk, chunk)]],
                   qrows.at[pl.ds(buf * chunk, chunk)])
            yield (tkv_hbm.at[idxkv.at[pl.ds(ci * chunk, chunk)]],
                   kvrows.at[pl.ds(buf * chunk, chunk)])

        def writes(ci, buf):
            yield (qrows.at[pl.ds(buf * chunk, chunk)],
                   gq_hbm.at[pl.ds(base + ci * chunk, chunk)])
            yield (kvrows.at[pl.ds(buf * chunk, chunk)],
                   gkv_hbm.at[pl.ds(base + ci * chunk, chunk)])

        def fire(ci, buf):
            for s, t in gathers(ci, buf):
                pltpu.async_copy(s, t, gsem)

        def drain_gather(ci, buf):
            for s, t in gathers(ci, buf):
                pltpu.make_async_copy(s, t, gsem).wait()

        def fire_write(ci, buf):
            for s, t in writes(ci, buf):
                pltpu.async_copy(s, t, wsem)

        def drain_write(ci, buf):
            for s, t in writes(ci, buf):
                pltpu.make_async_copy(s, t, wsem).wait()

        for j in range(min(3, nch)):
            fire(j, j)

        def body(ci, _):
            buf = lax.rem(ci, 4)
            pre = ci + 3
            pbuf = lax.rem(pre, 4)

            @pl.when(pre < nch)
            def _():
                @pl.when(ci >= 1)
                def _():
                    drain_write(ci - 1, lax.rem(ci - 1, 4))
                fire(pre, pbuf)

            drain_gather(ci, buf)
            fire_write(ci, buf)
            return 0

        lax.fori_loop(0, nch, body, 0)
        for j in range(max(0, nch - 4), nch):
            drain_write(j, j % 4)

    return k(tq, tkv, dst, src)


def _sc_scatter_add(wex, dst, zeros, n, chunk=80):
    e, w = wex.shape
    per_w = e // _NW
    nch = per_w // chunk
    nrow = (n // _NS) & ~7
    nlast = n - (_NS - 1) * nrow

    @functools.partial(
        pl.kernel,
        out_type=jax.ShapeDtypeStruct((_NC, n, w), jnp.float32),
        mesh=_sc_mesh(),
        compiler_params=pltpu.CompilerParams(use_tc_tiling_on_sc=False),
        scratch_types=[
            pltpu.VMEM((per_w,), jnp.int32),
            pltpu.VMEM((3 * chunk, w), jnp.float32),
            pltpu.VMEM_SHARED((n, w), jnp.float32),
            pltpu.SemaphoreType.DMA,
            pltpu.SemaphoreType.DMA,
        ],
    )
    def k(wex_hbm, dst_hbm, zero_hbm, out_hbm, idxv, rows, table, lsem, ssem):
        cid = lax.axis_index("c")
        sid = lax.axis_index("s")
        wid = sid * _NC + cid
        base = wid * per_w
        @pl.when(sid < _NS - 1)
        def _():
            pltpu.sync_copy(zero_hbm.at[pl.ds(sid * nrow, nrow)],
                            table.at[pl.ds(sid * nrow, nrow)])

        @pl.when(sid == _NS - 1)
        def _():
            pltpu.sync_copy(zero_hbm.at[pl.ds((_NS - 1) * nrow, nlast)],
                            table.at[pl.ds((_NS - 1) * nrow, nlast)])

        pltpu.sync_copy(dst_hbm.at[pl.ds(base, per_w)], idxv)
        plsc.subcore_barrier()

        def fire(ci, buf):
            return pltpu.async_copy(
                wex_hbm.at[pl.ds(base + ci * chunk, chunk)],
                rows.at[pl.ds(buf * chunk, chunk)], lsem)

        def drain(ci, buf):
            pltpu.make_async_copy(
                wex_hbm.at[pl.ds(base + ci * chunk, chunk)],
                rows.at[pl.ds(buf * chunk, chunk)], lsem).wait()

        def fire_scat(ci, buf):
            return pltpu.async_copy(
                rows.at[pl.ds(buf * chunk, chunk)],
                table.at[idxv.at[pl.ds(ci * chunk, chunk)]], ssem, add=True)

        def drain_scat(ci, buf):
            pltpu.make_async_copy(
                rows.at[pl.ds(buf * chunk, chunk)],
                table.at[idxv.at[pl.ds(ci * chunk, chunk)]], ssem).wait()

        for j in range(min(2, nch)):
            fire(j, j)

        def body(ci, _):
            buf = lax.rem(ci, 3)
            pre = ci + 2

            @pl.when(pre < nch)
            def _():
                @pl.when(ci >= 1)
                def _():
                    drain_scat(ci - 1, lax.rem(ci - 1, 3))
                fire(pre, lax.rem(pre, 3))

            drain(ci, buf)
            fire_scat(ci, buf)
            return 0

        lax.fori_loop(0, nch, body, 0)
        for j in range(max(0, nch - 3), nch):
            drain_scat(j, j % 3)
        plsc.subcore_barrier()

        @pl.when(sid < _NS - 1)
        def _():
            pltpu.sync_copy(table.at[pl.ds(sid * nrow, nrow)],
                            out_hbm.at[cid, pl.ds(sid * nrow, nrow)])

        @pl.when(sid == _NS - 1)
        def _():
            pltpu.sync_copy(table.at[pl.ds((_NS - 1) * nrow, nlast)],
                            out_hbm.at[cid, pl.ds((_NS - 1) * nrow, nlast)])

    return k(wex, dst, zeros)


def _ln(x, g, b, eps=1e-5):
    mu = x.mean(axis=-1, keepdims=True)
    var = ((x - mu) ** 2).mean(axis=-1, keepdims=True)
    return (x - mu) / jnp.sqrt(var + eps) * g + b


def _pack2(a, b):
    a16 = jax.lax.bitcast_convert_type(a.astype(jnp.bfloat16), jnp.uint16)
    b16 = jax.lax.bitcast_convert_type(b.astype(jnp.bfloat16), jnp.uint16)
    return (a16.astype(jnp.uint32)
            | (b16.astype(jnp.uint32) << 16)).astype(jnp.int32)


def _unpack2(w):
    wi = w.astype(jnp.uint32)
    a = jax.lax.bitcast_convert_type((wi << 16).astype(jnp.int32), jnp.float32)
    b = jax.lax.bitcast_convert_type(
        (wi & jnp.uint32(0xFFFF0000)).astype(jnp.int32), jnp.float32)
    return a, b


def _node_pre_body(x_ref, wqT_ref, wkvT_ref, bq_ref, bkv_ref, g1_ref, b1_ref,
                   h_ref, qn_ref, knv_ref):
    d = x_ref.shape[1]
    x = x_ref[...]
    h = _ln(x, g1_ref[...], b1_ref[...])
    h_ref[...] = h
    qn_ref[...] = jnp.dot(h, wqT_ref[...], preferred_element_type=jnp.float32) + bq_ref[...]
    knv = jnp.dot(h, wkvT_ref[...], preferred_element_type=jnp.float32) + bkv_ref[...]
    knv_ref[...] = _pack2(knv[:, :d], knv[:, d:])


def _node_pre(x, wqT, wkvT, bq, bkv, g1, b1, bn):
    n, d = x.shape
    grid = (n // bn,)
    f32 = jnp.float32
    return pl.pallas_call(
        _node_pre_body,
        grid=grid,
        in_specs=[
            pl.BlockSpec((bn, d), lambda i: (i, 0)),
            pl.BlockSpec(wqT.shape, lambda i: (0, 0)),
            pl.BlockSpec(wkvT.shape, lambda i: (0, 0)),
            pl.BlockSpec((1, d), lambda i: (0, 0)),
            pl.BlockSpec((1, 2 * d), lambda i: (0, 0)),
            pl.BlockSpec((1, d), lambda i: (0, 0)),
            pl.BlockSpec((1, d), lambda i: (0, 0)),
        ],
        out_specs=[
            pl.BlockSpec((bn, d), lambda i: (i, 0)),
            pl.BlockSpec((bn, d), lambda i: (i, 0)),
            pl.BlockSpec((bn, d), lambda i: (i, 0)),
        ],
        out_shape=[
            jax.ShapeDtypeStruct((n, d), f32),
            jax.ShapeDtypeStruct((n, d), f32),
            jax.ShapeDtypeStruct((n, d), jnp.int32),
        ],
    )(x, wqT, wkvT, bq, bkv, g1, b1)


def _edge_body(gq_ref, gkv_ref, ea_ref, wkvT_ref, bkv_ref, red_ref, exp_ref,
               out_ref):
    d = gq_ref.shape[1]
    kv = jnp.dot(ea_ref[...], wkvT_ref[...], preferred_element_type=jnp.float32) + bkv_ref[...]
    ke = kv[:, :d]
    ve = kv[:, d:]
    gkn, gvn = _unpack2(gkv_ref[...])
    prod = gq_ref[...] * (gkn + ke)
    alpha = jnp.dot(prod, red_ref[...], preferred_element_type=jnp.float32)
    ex = jnp.exp(alpha)
    exb = jnp.dot(ex, exp_ref[...], preferred_element_type=jnp.float32)
    w = (gvn + ve) * exb
    out_ref[...] = jnp.concatenate([w, ex], axis=1)


def _edge_stage(gq, gkv, ea, wkvT, bkv, red, expm, be):
    e, d = ea.shape
    grid = (e // be,)
    return pl.pallas_call(
        _edge_body,
        grid=grid,
        in_specs=[
            pl.BlockSpec((be, d), lambda i: (i, 0)),
            pl.BlockSpec((be, d), lambda i: (i, 0)),
            pl.BlockSpec((be, d), lambda i: (i, 0)),
            pl.BlockSpec(wkvT.shape, lambda i: (0, 0)),
            pl.BlockSpec((1, 2 * d), lambda i: (0, 0)),
            pl.BlockSpec(red.shape, lambda i: (0, 0)),
            pl.BlockSpec(expm.shape, lambda i: (0, 0)),
        ],
        out_specs=pl.BlockSpec((be, d + H), lambda i: (i, 0)),
        out_shape=jax.ShapeDtypeStruct((e, d + H), jnp.float32),
    )(gq, gkv, ea, wkvT, bkv, red, expm)


def _node_post_body(tbl_ref, x_ref, h_ref, expm_ref, wihT_ref, whhT_ref,
                    wselfT_ref, woutT_ref, b4_ref, g2_ref, b2_ref, wm1T_ref,
                    bm1_ref, wm2T_ref, bm2_ref, out_ref):
    d = x_ref.shape[1]
    t = jnp.sum(tbl_ref[...], axis=0)
    u = t[:, :d]
    den = t[:, d:]
    denb = jnp.dot(den, expm_ref[...], preferred_element_type=jnp.float32)
    agg = u / (denb + 1e-16)
    h = h_ref[...]
    b4 = b4_ref[...]
    gate = jax.nn.sigmoid(
        jnp.dot(agg, wihT_ref[...], preferred_element_type=jnp.float32)
        + jnp.dot(h, whhT_ref[...], preferred_element_type=jnp.float32)
        + b4[:, :d] + b4[:, d:2 * d])
    hs = jnp.dot(h, wselfT_ref[...], preferred_element_type=jnp.float32) + b4[:, 2 * d:3 * d]
    upd = agg + gate * (hs - agg)
    x1 = x_ref[...] + jnp.dot(upd, woutT_ref[...], preferred_element_type=jnp.float32) + b4[:, 3 * d:]
    h2 = _ln(x1, g2_ref[...], b2_ref[...])
    ff = jax.nn.relu(jnp.dot(h2, wm1T_ref[...], preferred_element_type=jnp.float32) + bm1_ref[...])
    ff = jnp.dot(ff, wm2T_ref[...], preferred_element_type=jnp.float32) + bm2_ref[...]
    out_ref[...] = x1 + ff


def _node_post(tbl, x, h, expm, wihT, whhT, wselfT, woutT, b4, g2, b2, wm1T,
               bm1, wm2T, bm2, bn):
    nsc, n, dh8 = tbl.shape
    d = x.shape[1]
    grid = (n // bn,)
    return pl.pallas_call(
        _node_post_body,
        grid=grid,
        in_specs=[
            pl.BlockSpec((nsc, bn, dh8), lambda i: (0, i, 0)),
            pl.BlockSpec((bn, d), lambda i: (i, 0)),
            pl.BlockSpec((bn, d), lambda i: (i, 0)),
            pl.BlockSpec(expm.shape, lambda i: (0, 0)),
            pl.BlockSpec(wihT.shape, lambda i: (0, 0)),
            pl.BlockSpec(whhT.shape, lambda i: (0, 0)),
            pl.BlockSpec(wselfT.shape, lambda i: (0, 0)),
            pl.BlockSpec(woutT.shape, lambda i: (0, 0)),
            pl.BlockSpec((1, 4 * d), lambda i: (0, 0)),
            pl.BlockSpec((1, d), lambda i: (0, 0)),
            pl.BlockSpec((1, d), lambda i: (0, 0)),
            pl.BlockSpec(wm1T.shape, lambda i: (0, 0)),
            pl.BlockSpec((1, 4 * d), lambda i: (0, 0)),
            pl.BlockSpec(wm2T.shape, lambda i: (0, 0)),
            pl.BlockSpec((1, d), lambda i: (0, 0)),
        ],
        out_specs=pl.BlockSpec((bn, d), lambda i: (i, 0)),
        out_shape=jax.ShapeDtypeStruct((n, d), jnp.float32),
    )(tbl, x, h, expm, wihT, whhT, wselfT, woutT, b4, g2, b2, wm1T, bm1, wm2T, bm2)


def kernel(x, edge_index, edge_attr, Wq, bq, Wkn, bkn, Wke, bke, Wvn, bvn,
           Wve, bve, Wself, bself, Wih, bih, Whh, bhh, Wout, bout, g1, b1,
           g2, b2, Wm1, bm1, Wm2, bm2):
    n, d = x.shape
    e = edge_index.shape[1]
    dh = d // H
    f32 = jnp.float32

    row = lambda v: v.reshape(1, -1)
    eye = jnp.eye(H, dtype=f32)
    red = jnp.repeat(eye, dh, axis=0) * (1.0 / (dh ** 0.5))
    expm = jnp.repeat(eye, dh, axis=1)

    bn = 1000 if n % 1000 == 0 else n
    be = 4000 if e % 4000 == 0 else e

    h, qn, knv = _node_pre(
        x, Wq.T, jnp.concatenate([Wkn.T, Wvn.T], axis=1), row(bq),
        row(jnp.concatenate([bkn, bvn])), row(g1), row(b1), bn)

    src, dst = _split_edges(edge_index)
    gq, gkv = _sc_gather2(qn, knv, src, dst)

    wex = _edge_stage(gq, gkv, edge_attr, jnp.concatenate([Wke.T, Wve.T], axis=1),
                      row(jnp.concatenate([bke, bve])), red, expm, be)

    zeros = jnp.zeros((n, d + H), f32)
    tbl = _sc_scatter_add(wex, dst, zeros, n)

    out = _node_post(
        tbl, x, h, expm, Wih.T, Whh.T, Wself.T, Wout.T,
        row(jnp.concatenate([bih, bhh, bself, bout])), row(g2), row(b2),
        Wm1.T, row(bm1), Wm2.T, row(bm2), bn)
    return out


# speedup vs baseline: 4.1591x; 1.3348x over previous
import functools

import jax
import jax.numpy as jnp
from jax import lax
from jax.experimental import pallas as pl
from jax.experimental.pallas import tpu as pltpu
from jax.experimental.pallas import tpu_sc as plsc

H = 8
_NC, _NS = 2, 16
_NW = _NC * _NS


def _sc_mesh():
    return plsc.VectorSubcoreMesh(core_axis_name="c", subcore_axis_name="s",
                                  num_cores=_NC, num_subcores=_NS)


def _split_edges(ei, bs=None):
    e = ei.shape[1]
    bs = e

    def body(ei_ref, s_ref, d_ref):
        s_ref[...] = ei_ref[0, :]
        d_ref[...] = ei_ref[1, :]

    return pl.pallas_call(
        body,
        grid=(e // bs,),
        in_specs=[pl.BlockSpec((2, bs), lambda i: (0, i))],
        out_specs=[pl.BlockSpec((bs,), lambda i: (i,)),
                   pl.BlockSpec((bs,), lambda i: (i,))],
        out_shape=[jax.ShapeDtypeStruct((e,), jnp.int32),
                   jax.ShapeDtypeStruct((e,), jnp.int32)],
    )(ei)


def _sc_gather2(tq, tkv, src, dst, chunk=80):
    n, wq = tq.shape
    wkv = tkv.shape[1]
    e = src.shape[0]
    per_w = e // _NW
    nch = per_w // chunk

    @functools.partial(
        pl.kernel,
        out_type=[jax.ShapeDtypeStruct((e, wq), jnp.float32),
                  jax.ShapeDtypeStruct((e, wkv), jnp.int32)],
        mesh=_sc_mesh(),
        scratch_types=[
            pltpu.VMEM((per_w,), jnp.int32),
            pltpu.VMEM((per_w,), jnp.int32),
            pltpu.VMEM((4 * chunk, wq), jnp.float32),
            pltpu.VMEM((4 * chunk, wkv), jnp.int32),
            pltpu.SemaphoreType.DMA,
            pltpu.SemaphoreType.DMA,
        ],
    )
    def k(tq_hbm, tkv_hbm, iq_hbm, ikv_hbm, gq_hbm, gkv_hbm,
          idxq, idxkv, qrows, kvrows, gsem, wsem):
        wid = lax.axis_index("s") * _NC + lax.axis_index("c")
        base = wid * per_w
        pltpu.sync_copy(iq_hbm.at[pl.ds(base, per_w)], idxq)
        pltpu.sync_copy(ikv_hbm.at[pl.ds(base, per_w)], idxkv)

        def gathers(ci, buf):
            yield (tq_hbm.at[idxq.at[pl.ds(ci * chunk, chunk)]],
                   qrows.at[pl.ds(buf * chunk, chunk)])
            yield (tkv_hbm.at[idxkv.at[pl.ds(ci * chunk, chunk)]],
                   kvrows.at[pl.ds(buf * chunk, chunk)])

        def writes(ci, buf):
            yield (qrows.at[pl.ds(buf * chunk, chunk)],
                   gq_hbm.at[pl.ds(base + ci * chunk, chunk)])
            yield (kvrows.at[pl.ds(buf * chunk, chunk)],
                   gkv_hbm.at[pl.ds(base + ci * chunk, chunk)])

        def fire(ci, buf):
            for s, t in gathers(ci, buf):
                pltpu.async_copy(s, t, gsem)

        def drain_gather(ci, buf):
            for s, t in gathers(ci, buf):
                pltpu.make_async_copy(s, t, gsem).wait()

        def fire_write(ci, buf):
            for s, t in writes(ci, buf):
                pltpu.async_copy(s, t, wsem)

        def drain_write(ci, buf):
            for s, t in writes(ci, buf):
                pltpu.make_async_copy(s, t, wsem).wait()

        for j in range(min(3, nch)):
            fire(j, j)

        def body(ci, _):
            buf = lax.rem(ci, 4)
            pre = ci + 3
            pbuf = lax.rem(pre, 4)

            @pl.when(pre < nch)
            def _():
                @pl.when(ci >= 1)
                def _():
                    drain_write(ci - 1, lax.rem(ci - 1, 4))
                fire(pre, pbuf)

            drain_gather(ci, buf)
            fire_write(ci, buf)
            return 0

        lax.fori_loop(0, nch, body, 0)
        for j in range(max(0, nch - 4), nch):
            drain_write(j, j % 4)

    return k(tq, tkv, dst, src)


def _sc_scatter_add(wex, dst, zeros, n, chunk=80, tc_tiling=True):
    e, w = wex.shape
    per_w = e // _NW
    nch = per_w // chunk
    nrow = (n // _NS) & ~7
    nlast = n - (_NS - 1) * nrow

    @functools.partial(
        pl.kernel,
        out_type=jax.ShapeDtypeStruct((_NC, n, w), jnp.float32),
        mesh=_sc_mesh(),
        compiler_params=None if tc_tiling else pltpu.CompilerParams(
            use_tc_tiling_on_sc=False),
        scratch_types=[
            pltpu.VMEM((per_w,), jnp.int32),
            pltpu.VMEM((3 * chunk, w), jnp.float32),
            pltpu.VMEM_SHARED((n, w), jnp.float32),
            pltpu.SemaphoreType.DMA,
            pltpu.SemaphoreType.DMA,
        ],
    )
    def k(wex_hbm, dst_hbm, zero_hbm, out_hbm, idxv, rows, table, lsem, ssem):
        cid = lax.axis_index("c")
        sid = lax.axis_index("s")
        wid = sid * _NC + cid
        base = wid * per_w
        @pl.when(sid < _NS - 1)
        def _():
            pltpu.sync_copy(zero_hbm.at[pl.ds(sid * nrow, nrow)],
                            table.at[pl.ds(sid * nrow, nrow)])

        @pl.when(sid == _NS - 1)
        def _():
            pltpu.sync_copy(zero_hbm.at[pl.ds((_NS - 1) * nrow, nlast)],
                            table.at[pl.ds((_NS - 1) * nrow, nlast)])

        pltpu.sync_copy(dst_hbm.at[pl.ds(base, per_w)], idxv)
        plsc.subcore_barrier()

        def fire(ci, buf):
            return pltpu.async_copy(
                wex_hbm.at[pl.ds(base + ci * chunk, chunk)],
                rows.at[pl.ds(buf * chunk, chunk)], lsem)

        def drain(ci, buf):
            pltpu.make_async_copy(
                wex_hbm.at[pl.ds(base + ci * chunk, chunk)],
                rows.at[pl.ds(buf * chunk, chunk)], lsem).wait()

        def fire_scat(ci, buf):
            return pltpu.async_copy(
                rows.at[pl.ds(buf * chunk, chunk)],
                table.at[idxv.at[pl.ds(ci * chunk, chunk)]], ssem, add=True)

        def drain_scat(ci, buf):
            pltpu.make_async_copy(
                rows.at[pl.ds(buf * chunk, chunk)],
                table.at[idxv.at[pl.ds(ci * chunk, chunk)]], ssem).wait()

        for j in range(min(2, nch)):
            fire(j, j)

        def body(ci, _):
            buf = lax.rem(ci, 3)
            pre = ci + 2

            @pl.when(pre < nch)
            def _():
                @pl.when(ci >= 1)
                def _():
                    drain_scat(ci - 1, lax.rem(ci - 1, 3))
                fire(pre, lax.rem(pre, 3))

            drain(ci, buf)
            fire_scat(ci, buf)
            return 0

        lax.fori_loop(0, nch, body, 0)
        for j in range(max(0, nch - 3), nch):
            drain_scat(j, j % 3)
        plsc.subcore_barrier()

        @pl.when(sid < _NS - 1)
        def _():
            pltpu.sync_copy(table.at[pl.ds(sid * nrow, nrow)],
                            out_hbm.at[cid, pl.ds(sid * nrow, nrow)])

        @pl.when(sid == _NS - 1)
        def _():
            pltpu.sync_copy(table.at[pl.ds((_NS - 1) * nrow, nlast)],
                            out_hbm.at[cid, pl.ds((_NS - 1) * nrow, nlast)])

    return k(wex, dst, zeros)


def _ln(x, g, b, eps=1e-5):
    mu = x.mean(axis=-1, keepdims=True)
    var = ((x - mu) ** 2).mean(axis=-1, keepdims=True)
    return (x - mu) / jnp.sqrt(var + eps) * g + b


def _pack2(a, b):
    a16 = jax.lax.bitcast_convert_type(a.astype(jnp.bfloat16), jnp.uint16)
    b16 = jax.lax.bitcast_convert_type(b.astype(jnp.bfloat16), jnp.uint16)
    return (a16.astype(jnp.uint32)
            | (b16.astype(jnp.uint32) << 16)).astype(jnp.int32)


def _unpack2(w):
    wi = w.astype(jnp.uint32)
    a = jax.lax.bitcast_convert_type((wi << 16).astype(jnp.int32), jnp.float32)
    b = jax.lax.bitcast_convert_type(
        (wi & jnp.uint32(0xFFFF0000)).astype(jnp.int32), jnp.float32)
    return a, b


def _node_pre_body(x_ref, wqT_ref, wkvT_ref, bq_ref, bkv_ref, g1_ref, b1_ref,
                   h_ref, qn_ref, knv_ref):
    d = x_ref.shape[1]
    x = x_ref[...]
    h = _ln(x, g1_ref[...], b1_ref[...])
    h_ref[...] = h
    qn_ref[...] = jnp.dot(h, wqT_ref[...], preferred_element_type=jnp.float32) + bq_ref[...]
    knv = jnp.dot(h, wkvT_ref[...], preferred_element_type=jnp.float32) + bkv_ref[...]
    knv_ref[...] = _pack2(knv[:, :d], knv[:, d:])


def _node_pre(x, wqT, wkvT, bq, bkv, g1, b1, bn):
    n, d = x.shape
    grid = (n // bn,)
    f32 = jnp.float32
    return pl.pallas_call(
        _node_pre_body,
        grid=grid,
        in_specs=[
            pl.BlockSpec((bn, d), lambda i: (i, 0)),
            pl.BlockSpec(wqT.shape, lambda i: (0, 0)),
            pl.BlockSpec(wkvT.shape, lambda i: (0, 0)),
            pl.BlockSpec((1, d), lambda i: (0, 0)),
            pl.BlockSpec((1, 2 * d), lambda i: (0, 0)),
            pl.BlockSpec((1, d), lambda i: (0, 0)),
            pl.BlockSpec((1, d), lambda i: (0, 0)),
        ],
        out_specs=[
            pl.BlockSpec((bn, d), lambda i: (i, 0)),
            pl.BlockSpec((bn, d), lambda i: (i, 0)),
            pl.BlockSpec((bn, d), lambda i: (i, 0)),
        ],
        out_shape=[
            jax.ShapeDtypeStruct((n, d), f32),
            jax.ShapeDtypeStruct((n, d), f32),
            jax.ShapeDtypeStruct((n, d), jnp.int32),
        ],
    )(x, wqT, wkvT, bq, bkv, g1, b1)


def _edge_body(gq_ref, gkv_ref, ea_ref, wkvT_ref, bkv_ref, red_ref, exp_ref,
               w_ref, ex_ref):
    d = gq_ref.shape[1]
    kv = jnp.dot(ea_ref[...], wkvT_ref[...], preferred_element_type=jnp.float32) + bkv_ref[...]
    ke = kv[:, :d]
    ve = kv[:, d:]
    gkn, gvn = _unpack2(gkv_ref[...])
    prod = gq_ref[...] * (gkn + ke)
    alpha = jnp.dot(prod, red_ref[...], preferred_element_type=jnp.float32)
    ex = jnp.exp(alpha)
    exb = jnp.dot(ex, exp_ref[...], preferred_element_type=jnp.float32)
    w_ref[...] = (gvn + ve) * exb
    ex_ref[...] = ex


def _edge_stage(gq, gkv, ea, wkvT, bkv, red, expm, be):
    e, d = ea.shape
    grid = (e // be,)
    return pl.pallas_call(
        _edge_body,
        grid=grid,
        in_specs=[
            pl.BlockSpec((be, d), lambda i: (i, 0)),
            pl.BlockSpec((be, d), lambda i: (i, 0)),
            pl.BlockSpec((be, d), lambda i: (i, 0)),
            pl.BlockSpec(wkvT.shape, lambda i: (0, 0)),
            pl.BlockSpec((1, 2 * d), lambda i: (0, 0)),
            pl.BlockSpec(red.shape, lambda i: (0, 0)),
            pl.BlockSpec(expm.shape, lambda i: (0, 0)),
        ],
        out_specs=[pl.BlockSpec((be, d), lambda i: (i, 0)),
                   pl.BlockSpec((be, H), lambda i: (i, 0))],
        out_shape=[jax.ShapeDtypeStruct((e, d), jnp.float32),
                   jax.ShapeDtypeStruct((e, H), jnp.float32)],
    )(gq, gkv, ea, wkvT, bkv, red, expm)


def _node_post_body(utbl_ref, dtbl_ref, x_ref, h_ref, expm_ref, wihT_ref,
                    whhT_ref, wselfT_ref, woutT_ref, b4_ref, g2_ref, b2_ref,
                    wm1T_ref, bm1_ref, wm2T_ref, bm2_ref, out_ref):
    d = x_ref.shape[1]
    u = jnp.sum(utbl_ref[...], axis=0)
    den = jnp.sum(dtbl_ref[...], axis=0)
    denb = jnp.dot(den, expm_ref[...], preferred_element_type=jnp.float32)
    agg = u / (denb + 1e-16)
    h = h_ref[...]
    b4 = b4_ref[...]
    gate = jax.nn.sigmoid(
        jnp.dot(agg, wihT_ref[...], preferred_element_type=jnp.float32)
        + jnp.dot(h, whhT_ref[...], preferred_element_type=jnp.float32)
        + b4[:, :d] + b4[:, d:2 * d])
    hs = jnp.dot(h, wselfT_ref[...], preferred_element_type=jnp.float32) + b4[:, 2 * d:3 * d]
    upd = agg + gate * (hs - agg)
    x1 = x_ref[...] + jnp.dot(upd, woutT_ref[...], preferred_element_type=jnp.float32) + b4[:, 3 * d:]
    h2 = _ln(x1, g2_ref[...], b2_ref[...])
    ff = jax.nn.relu(jnp.dot(h2, wm1T_ref[...], preferred_element_type=jnp.float32) + bm1_ref[...])
    ff = jnp.dot(ff, wm2T_ref[...], preferred_element_type=jnp.float32) + bm2_ref[...]
    out_ref[...] = x1 + ff


def _node_post(utbl, dtbl, x, h, expm, wihT, whhT, wselfT, woutT, b4, g2, b2,
               wm1T, bm1, wm2T, bm2, bn):
    nsc = utbl.shape[0]
    d = x.shape[1]
    n = x.shape[0]
    grid = (n // bn,)
    return pl.pallas_call(
        _node_post_body,
        grid=grid,
        in_specs=[
            pl.BlockSpec((nsc, bn, d), lambda i: (0, i, 0)),
            pl.BlockSpec((nsc, bn, H), lambda i: (0, i, 0)),
            pl.BlockSpec((bn, d), lambda i: (i, 0)),
            pl.BlockSpec((bn, d), lambda i: (i, 0)),
            pl.BlockSpec(expm.shape, lambda i: (0, 0)),
            pl.BlockSpec(wihT.shape, lambda i: (0, 0)),
            pl.BlockSpec(whhT.shape, lambda i: (0, 0)),
            pl.BlockSpec(wselfT.shape, lambda i: (0, 0)),
            pl.BlockSpec(woutT.shape, lambda i: (0, 0)),
            pl.BlockSpec((1, 4 * d), lambda i: (0, 0)),
            pl.BlockSpec((1, d), lambda i: (0, 0)),
            pl.BlockSpec((1, d), lambda i: (0, 0)),
            pl.BlockSpec(wm1T.shape, lambda i: (0, 0)),
            pl.BlockSpec((1, 4 * d), lambda i: (0, 0)),
            pl.BlockSpec(wm2T.shape, lambda i: (0, 0)),
            pl.BlockSpec((1, d), lambda i: (0, 0)),
        ],
        out_specs=pl.BlockSpec((bn, d), lambda i: (i, 0)),
        out_shape=jax.ShapeDtypeStruct((n, d), jnp.float32),
    )(utbl, dtbl, x, h, expm, wihT, whhT, wselfT, woutT, b4, g2, b2, wm1T,
      bm1, wm2T, bm2)


def kernel(x, edge_index, edge_attr, Wq, bq, Wkn, bkn, Wke, bke, Wvn, bvn,
           Wve, bve, Wself, bself, Wih, bih, Whh, bhh, Wout, bout, g1, b1,
           g2, b2, Wm1, bm1, Wm2, bm2):
    n, d = x.shape
    e = edge_index.shape[1]
    dh = d // H
    f32 = jnp.float32

    row = lambda v: v.reshape(1, -1)
    eye = jnp.eye(H, dtype=f32)
    red = jnp.repeat(eye, dh, axis=0) * (1.0 / (dh ** 0.5))
    expm = jnp.repeat(eye, dh, axis=1)

    bn = 1000 if n % 1000 == 0 else n
    be = 4000 if e % 4000 == 0 else e

    h, qn, knv = _node_pre(
        x, Wq.T, jnp.concatenate([Wkn.T, Wvn.T], axis=1), row(bq),
        row(jnp.concatenate([bkn, bvn])), row(g1), row(b1), bn)

    src, dst = _split_edges(edge_index)
    gq, gkv = _sc_gather2(qn, knv, src, dst)

    w, ex = _edge_stage(gq, gkv, edge_attr, jnp.concatenate([Wke.T, Wve.T], axis=1),
                        row(jnp.concatenate([bke, bve])), red, expm, be)

    utbl = _sc_scatter_add(w, dst, jnp.zeros((n, d), f32), n)
    dtbl = _sc_scatter_add(ex, dst, jnp.zeros((n, H), f32), n, tc_tiling=False)

    out = _node_post(
        utbl, dtbl, x, h, expm, Wih.T, Whh.T, Wself.T, Wout.T,
        row(jnp.concatenate([bih, bhh, bself, bout])), row(g2), row(b2),
        Wm1.T, row(bm1), Wm2.T, row(bm2), bn)
    return out
